# baseline (speedup 1.0000x reference)
import functools

import jax
import jax.numpy as jnp
from jax import lax
from jax.experimental import pallas as pl
from jax.experimental.pallas import tpu as pltpu
from jax.experimental.pallas import tpu_sc as plsc

N = 10000
NP = 10240
E = 320000
D = 128
NC = 2
NS = 16
NW = NC * NS
EW = E // NW
SEG = NP // NS
R, C = 125, 80
P = 4

_mesh = plsc.VectorSubcoreMesh(core_axis_name="c", subcore_axis_name="s")
_params = pltpu.CompilerParams(needs_layout_passes=False)


def _zero_fill(ref, words):
    for k in range(words // 16):
        ref[pl.ds(k * 16, 16)] = jnp.zeros((16,), jnp.float32)


def _acc_seg(seg_v, tmp_v, parts, sl):
    pltpu.sync_copy(parts[0].at[sl], seg_v)
    for p in range(1, P):
        pltpu.sync_copy(parts[p].at[sl], tmp_v)
        for k in range(SEG // 16):
            s = pl.ds(k * 16, 16)
            seg_v[s] = seg_v[s] + tmp_v[s]


@functools.partial(
    pl.kernel,
    out_type=[jax.ShapeDtypeStruct((NP,), jnp.float32) for _ in range(4)],
    mesh=_mesh,
    compiler_params=_params,
    scratch_types=[
        pltpu.VMEM((R, C), jnp.int32),
        pltpu.VMEM((R, C), jnp.int32),
        pltpu.VMEM((C,), jnp.float32),
        pltpu.VMEM((SEG,), jnp.float32),
        pltpu.VMEM((SEG,), jnp.float32),
    ]
    + [pltpu.VMEM_SHARED((NP,), jnp.float32) for _ in range(2 * P)]
    + [pltpu.SemaphoreType.DMA],
)
def _k1(es_hbm, do0_hbm, di0_hbm, do1_hbm, di1_hbm,
        src_v, dst_v, ones_v, seg_v, tmp_v, *rest):
    do_sp = rest[:P]
    di_sp = rest[P:2 * P]
    sem = rest[2 * P]
    cid = lax.axis_index("c")
    sid = lax.axis_index("s")
    wid = sid * NC + cid

    _zero_fill(seg_v, SEG)
    for k in range(C // 16):
        ones_v[pl.ds(k * 16, 16)] = jnp.ones((16,), jnp.float32)

    pltpu.sync_copy(es_hbm.at[0, wid], src_v)
    pltpu.sync_copy(es_hbm.at[1, wid], dst_v)
    for p in range(P):
        pltpu.sync_copy(seg_v, do_sp[p].at[pl.ds(sid * SEG, SEG)])
        pltpu.sync_copy(seg_v, di_sp[p].at[pl.ds(sid * SEG, SEG)])
    plsc.subcore_barrier()

    def quad(t, _):
        @pl.when(t > 0)
        def _():
            pltpu.make_async_copy(do0_hbm.at[pl.ds(0, SEG)], seg_v,
                                  sem).wait()

        j0 = t * P
        for p in range(P):
            pltpu.async_copy(ones_v, do_sp[p].at[src_v.at[j0 + p]], sem,
                             add=True)
            pltpu.async_copy(ones_v, di_sp[p].at[dst_v.at[j0 + p]], sem,
                             add=True)
        return 0

    NQ = R // P
    lax.fori_loop(0, NQ, quad, 0)
    pltpu.make_async_copy(do0_hbm.at[pl.ds(0, SEG)], seg_v, sem).wait()
    for j in range(NQ * P, R):
        pltpu.async_copy(ones_v, do_sp[0].at[src_v.at[j]], sem, add=True)
        pltpu.async_copy(ones_v, di_sp[0].at[dst_v.at[j]], sem, add=True)
        pltpu.make_async_copy(do0_hbm.at[pl.ds(0, 2 * C)],
                              seg_v.at[pl.ds(0, 2 * C)], sem).wait()
    plsc.subcore_barrier()

    sl = pl.ds(sid * SEG, SEG)

    @pl.when(cid == 0)
    def _():
        _acc_seg(seg_v, tmp_v, do_sp, sl)
        pltpu.sync_copy(seg_v, do0_hbm.at[sl])
        _acc_seg(seg_v, tmp_v, di_sp, sl)
        pltpu.sync_copy(seg_v, di0_hbm.at[sl])

    @pl.when(cid == 1)
    def _():
        _acc_seg(seg_v, tmp_v, do_sp, sl)
        pltpu.sync_copy(seg_v, do1_hbm.at[sl])
        _acc_seg(seg_v, tmp_v, di_sp, sl)
        pltpu.sync_copy(seg_v, di1_hbm.at[sl])


def _k2_body(x_ref, w_ref, do0_ref, di0_ref, do1_ref, di1_ref,
             h_ref, nd_ref):
    xw = jnp.sum(x_ref[...] * w_ref[...], axis=-1)
    deg_out = do0_ref[pl.ds(0, N)] + do1_ref[pl.ds(0, N)]
    deg_in = di0_ref[pl.ds(0, N)] + di1_ref[pl.ds(0, N)]
    h_ref[...] = xw * lax.rsqrt(jnp.maximum(deg_out, 1.0))
    nd_ref[...] = lax.rsqrt(jnp.maximum(deg_in, 1.0))


_k2 = pl.pallas_call(
    _k2_body,
    out_shape=(
        jax.ShapeDtypeStruct((N,), jnp.float32),
        jax.ShapeDtypeStruct((N,), jnp.float32),
    ),
)


@functools.partial(
    pl.kernel,
    out_type=[jax.ShapeDtypeStruct((NP,), jnp.float32) for _ in range(2)],
    mesh=_mesh,
    compiler_params=_params,
    scratch_types=[
        pltpu.VMEM((R, C), jnp.int32),
        pltpu.VMEM((R, C), jnp.int32),
        pltpu.VMEM((R, C), jnp.float32),
        pltpu.VMEM((N,), jnp.float32),
        pltpu.VMEM((SEG,), jnp.float32),
        pltpu.VMEM((SEG,), jnp.float32),
    ]
    + [pltpu.VMEM_SHARED((NP,), jnp.float32) for _ in range(P)]
    + [pltpu.SemaphoreType.DMA],
)
def _k3(es_hbm, h_hbm, a0_hbm, a1_hbm,
        src_v, dst_v, vals_v, h_v, seg_v, tmp_v, *rest):
    agg_sp = rest[:P]
    sem = rest[P]
    cid = lax.axis_index("c")
    sid = lax.axis_index("s")
    wid = sid * NC + cid

    _zero_fill(seg_v, SEG)
    pltpu.sync_copy(es_hbm.at[0, wid], src_v)
    pltpu.sync_copy(es_hbm.at[1, wid], dst_v)
    pltpu.sync_copy(h_hbm, h_v)
    for p in range(P):
        pltpu.sync_copy(seg_v, agg_sp[p].at[pl.ds(sid * SEG, SEG)])
    plsc.subcore_barrier()

    def quad(t, _):
        @pl.when(t > 0)
        def _():
            pltpu.make_async_copy(h_hbm.at[pl.ds(0, P * C)],
                                  seg_v.at[pl.ds(0, P * C)], sem).wait()

        j0 = t * P
        for p in range(P):
            j = j0 + p
            for k in range(C // 16):
                idx16 = src_v[j, pl.ds(k * 16, 16)]
                vals_v[j, pl.ds(k * 16, 16)] = plsc.load_gather(h_v, [idx16])
            pltpu.async_copy(vals_v.at[j], agg_sp[p].at[dst_v.at[j]], sem,
                             add=True)
        return 0

    NQ = R // P
    lax.fori_loop(0, NQ, quad, 0)
    pltpu.make_async_copy(h_hbm.at[pl.ds(0, P * C)],
                          seg_v.at[pl.ds(0, P * C)], sem).wait()
    for j in range(NQ * P, R):
        for k in range(C // 16):
            idx16 = src_v[j, pl.ds(k * 16, 16)]
            vals_v[j, pl.ds(k * 16, 16)] = plsc.load_gather(h_v, [idx16])
        pltpu.async_copy(vals_v.at[j], agg_sp[0].at[dst_v.at[j]], sem,
                         add=True)
        pltpu.make_async_copy(h_hbm.at[pl.ds(0, C)],
                              seg_v.at[pl.ds(0, C)], sem).wait()
    plsc.subcore_barrier()

    sl = pl.ds(sid * SEG, SEG)
    _acc_seg(seg_v, tmp_v, agg_sp, sl)

    @pl.when(cid == 0)
    def _():
        pltpu.sync_copy(seg_v, a0_hbm.at[sl])

    @pl.when(cid == 1)
    def _():
        pltpu.sync_copy(seg_v, a1_hbm.at[sl])


def _k4_body(a0_ref, a1_ref, nd_ref, b_ref, out_ref):
    agg = a0_ref[pl.ds(0, N)] + a1_ref[pl.ds(0, N)]
    o = jnp.maximum(agg * nd_ref[...] + b_ref[0, 0], 0.0)
    out_ref[...] = o.reshape(N, 1)


_k4 = pl.pallas_call(
    _k4_body,
    out_shape=jax.ShapeDtypeStruct((N, 1), jnp.float32),
)


def kernel(x, edge_index, W_mat, b):
    es = edge_index.reshape(2, NW, R, C)
    wr = W_mat.reshape(1, D)

    do0, di0, do1, di1 = _k1(es)
    h, nd = _k2(x, wr, do0, di0, do1, di1)

    a0, a1 = _k3(es, h)
    return _k4(a0, a1, nd, b.reshape(1, 1))


# speedup vs baseline: 1.0926x; 1.0926x over previous
import functools

import jax
import jax.numpy as jnp
from jax import lax
from jax.experimental import pallas as pl
from jax.experimental.pallas import tpu as pltpu
from jax.experimental.pallas import tpu_sc as plsc

N = 10000
NP = 10240
E = 320000
D = 128
NC = 2
NS = 16
NW = NC * NS
EW = E // NW
SEG = NP // NS
R, C = 125, 80
P = 4

_mesh = plsc.VectorSubcoreMesh(core_axis_name="c", subcore_axis_name="s")
_params = pltpu.CompilerParams(needs_layout_passes=False)


def _zero_fill(ref, words):
    for k in range(words // 16):
        ref[pl.ds(k * 16, 16)] = jnp.zeros((16,), jnp.float32)


def _acc_seg(seg_v, tmp_v, parts, sl, n=None):
    n = len(parts) if n is None else n
    pltpu.sync_copy(parts[0].at[sl], seg_v)
    for p in range(1, n):
        pltpu.sync_copy(parts[p].at[sl], tmp_v)
        for k in range(SEG // 16):
            s = pl.ds(k * 16, 16)
            seg_v[s] = seg_v[s] + tmp_v[s]


@functools.partial(
    pl.kernel,
    out_type=[jax.ShapeDtypeStruct((NP,), jnp.float32) for _ in range(4)],
    mesh=_mesh,
    compiler_params=_params,
    scratch_types=[
        pltpu.VMEM((R, C), jnp.int32),
        pltpu.VMEM((R, C), jnp.int32),
        pltpu.VMEM((C,), jnp.float32),
        pltpu.VMEM((SEG,), jnp.float32),
        pltpu.VMEM((SEG,), jnp.float32),
    ]
    + [pltpu.VMEM_SHARED((NP,), jnp.float32) for _ in range(2 * P)]
    + [pltpu.SemaphoreType.DMA],
)
def _k1(es_hbm, do0_hbm, di0_hbm, do1_hbm, di1_hbm,
        src_v, dst_v, ones_v, seg_v, tmp_v, *rest):
    do_sp = rest[:P]
    di_sp = rest[P:2 * P]
    sem = rest[2 * P]
    cid = lax.axis_index("c")
    sid = lax.axis_index("s")
    wid = sid * NC + cid

    _zero_fill(seg_v, SEG)
    for k in range(C // 16):
        ones_v[pl.ds(k * 16, 16)] = jnp.ones((16,), jnp.float32)

    pltpu.sync_copy(es_hbm.at[0, wid], src_v)
    pltpu.sync_copy(es_hbm.at[1, wid], dst_v)
    for p in range(P):
        pltpu.sync_copy(seg_v, do_sp[p].at[pl.ds(sid * SEG, SEG)])
        pltpu.sync_copy(seg_v, di_sp[p].at[pl.ds(sid * SEG, SEG)])
    plsc.subcore_barrier()

    def quad(t, _):
        @pl.when(t > 0)
        def _():
            pltpu.make_async_copy(do0_hbm.at[pl.ds(0, SEG)], seg_v,
                                  sem).wait()

        j0 = t * P
        for p in range(P):
            pltpu.async_copy(ones_v, do_sp[p].at[src_v.at[j0 + p]], sem,
                             add=True)
            pltpu.async_copy(ones_v, di_sp[p].at[dst_v.at[j0 + p]], sem,
                             add=True)
        return 0

    NQ = R // P
    lax.fori_loop(0, NQ, quad, 0)
    pltpu.make_async_copy(do0_hbm.at[pl.ds(0, SEG)], seg_v, sem).wait()
    for j in range(NQ * P, R):
        pltpu.async_copy(ones_v, do_sp[0].at[src_v.at[j]], sem, add=True)
        pltpu.async_copy(ones_v, di_sp[0].at[dst_v.at[j]], sem, add=True)
        pltpu.make_async_copy(do0_hbm.at[pl.ds(0, 2 * C)],
                              seg_v.at[pl.ds(0, 2 * C)], sem).wait()
    plsc.subcore_barrier()

    sl = pl.ds(sid * SEG, SEG)

    @pl.when(cid == 0)
    def _():
        _acc_seg(seg_v, tmp_v, do_sp, sl)
        pltpu.sync_copy(seg_v, do0_hbm.at[sl])
        _acc_seg(seg_v, tmp_v, di_sp, sl)
        pltpu.sync_copy(seg_v, di0_hbm.at[sl])

    @pl.when(cid == 1)
    def _():
        _acc_seg(seg_v, tmp_v, do_sp, sl)
        pltpu.sync_copy(seg_v, do1_hbm.at[sl])
        _acc_seg(seg_v, tmp_v, di_sp, sl)
        pltpu.sync_copy(seg_v, di1_hbm.at[sl])


def _k2a_body(x_ref, w_ref, xw_ref):
    xw_ref[...] = jnp.sum(x_ref[...] * w_ref[...], axis=-1)


_k2a = pl.pallas_call(
    _k2a_body,
    out_shape=jax.ShapeDtypeStruct((N,), jnp.float32),
)


def _k2b_body(xw_ref, do0_ref, di0_ref, do1_ref, di1_ref, h_ref, nd_ref):
    deg_out = do0_ref[pl.ds(0, N)] + do1_ref[pl.ds(0, N)]
    deg_in = di0_ref[pl.ds(0, N)] + di1_ref[pl.ds(0, N)]
    h_ref[...] = xw_ref[...] * lax.rsqrt(jnp.maximum(deg_out, 1.0))
    nd_ref[...] = lax.rsqrt(jnp.maximum(deg_in, 1.0))


_k2b = pl.pallas_call(
    _k2b_body,
    out_shape=(
        jax.ShapeDtypeStruct((N,), jnp.float32),
        jax.ShapeDtypeStruct((N,), jnp.float32),
    ),
)


@functools.partial(
    pl.kernel,
    out_type=[jax.ShapeDtypeStruct((NP,), jnp.float32) for _ in range(2)],
    mesh=_mesh,
    compiler_params=_params,
    scratch_types=[
        pltpu.VMEM((R, C), jnp.int32),
        pltpu.VMEM((R, C), jnp.int32),
        pltpu.VMEM((R, C), jnp.float32),
        pltpu.VMEM((N,), jnp.float32),
        pltpu.VMEM((SEG,), jnp.float32),
        pltpu.VMEM((SEG,), jnp.float32),
    ]
    + [pltpu.VMEM_SHARED((NP,), jnp.float32) for _ in range(2 * P)]
    + [pltpu.SemaphoreType.DMA],
)
def _k3(es_hbm, h_hbm, a0_hbm, a1_hbm,
        src_v, dst_v, vals_v, h_v, seg_v, tmp_v, *rest):
    agg_sp = rest[:2 * P]
    sem = rest[2 * P]
    cid = lax.axis_index("c")
    sid = lax.axis_index("s")
    wid = sid * NC + cid

    _zero_fill(seg_v, SEG)
    pltpu.sync_copy(es_hbm.at[0, wid], src_v)
    pltpu.sync_copy(es_hbm.at[1, wid], dst_v)
    pltpu.sync_copy(h_hbm, h_v)
    for p in range(2 * P):
        pltpu.sync_copy(seg_v, agg_sp[p].at[pl.ds(sid * SEG, SEG)])
    plsc.subcore_barrier()

    def quad(t, _):
        @pl.when(t > 1)
        def _():
            pltpu.make_async_copy(h_hbm.at[pl.ds(0, P * C)],
                                  seg_v.at[pl.ds(0, P * C)], sem).wait()

        j0 = t * P
        grp = lax.rem(t, 2) * P
        for p in range(P):
            j = j0 + p
            for k in range(C // 16):
                idx16 = src_v[j, pl.ds(k * 16, 16)]
                vals_v[j, pl.ds(k * 16, 16)] = plsc.load_gather(h_v, [idx16])

            @pl.when(grp == 0)
            def _(p=p, j=j):
                pltpu.async_copy(vals_v.at[j], agg_sp[p].at[dst_v.at[j]],
                                 sem, add=True)

            @pl.when(grp == P)
            def _(p=p, j=j):
                pltpu.async_copy(vals_v.at[j], agg_sp[P + p].at[dst_v.at[j]],
                                 sem, add=True)
        return 0

    NQ = R // P
    lax.fori_loop(0, NQ, quad, 0)
    pltpu.make_async_copy(h_hbm.at[pl.ds(0, 2 * P * C)],
                          seg_v.at[pl.ds(0, 2 * P * C)], sem).wait()
    for j in range(NQ * P, R):
        for k in range(C // 16):
            idx16 = src_v[j, pl.ds(k * 16, 16)]
            vals_v[j, pl.ds(k * 16, 16)] = plsc.load_gather(h_v, [idx16])
        pltpu.async_copy(vals_v.at[j], agg_sp[0].at[dst_v.at[j]], sem,
                         add=True)
        pltpu.make_async_copy(h_hbm.at[pl.ds(0, C)],
                              seg_v.at[pl.ds(0, C)], sem).wait()
    plsc.subcore_barrier()

    sl = pl.ds(sid * SEG, SEG)
    _acc_seg(seg_v, tmp_v, agg_sp, sl, 2 * P)

    @pl.when(cid == 0)
    def _():
        pltpu.sync_copy(seg_v, a0_hbm.at[sl])

    @pl.when(cid == 1)
    def _():
        pltpu.sync_copy(seg_v, a1_hbm.at[sl])


def _k4_body(a0_ref, a1_ref, nd_ref, b_ref, out_ref):
    agg = a0_ref[pl.ds(0, N)] + a1_ref[pl.ds(0, N)]
    o = jnp.maximum(agg * nd_ref[...] + b_ref[0, 0], 0.0)
    out_ref[...] = o.reshape(N, 1)


_k4 = pl.pallas_call(
    _k4_body,
    out_shape=jax.ShapeDtypeStruct((N, 1), jnp.float32),
)


def kernel(x, edge_index, W_mat, b):
    es = edge_index.reshape(2, NW, R, C)
    wr = W_mat.reshape(1, D)

    xw = _k2a(x, wr)
    do0, di0, do1, di1 = _k1(es)
    h, nd = _k2b(xw, do0, di0, do1, di1)

    a0, a1 = _k3(es, h)
    return _k4(a0, a1, nd, b.reshape(1, 1))


# speedup vs baseline: 1.1925x; 1.0915x over previous
import functools

import jax
import jax.numpy as jnp
from jax import lax
from jax.experimental import pallas as pl
from jax.experimental.pallas import tpu as pltpu
from jax.experimental.pallas import tpu_sc as plsc

N = 10000
NP = 10240
E = 320000
D = 128
NC = 2
NS = 16
NW = NC * NS
EW = E // NW
SEG = NP // NS
R, C = 125, 80
P = 4

_mesh = plsc.VectorSubcoreMesh(core_axis_name="c", subcore_axis_name="s")
_params = pltpu.CompilerParams(needs_layout_passes=False)


def _zero_fill(ref, words):
    for k in range(words // 16):
        ref[pl.ds(k * 16, 16)] = jnp.zeros((16,), jnp.float32)


def _acc_seg(seg_v, tmp_v, parts, sl, n=None):
    n = len(parts) if n is None else n
    pltpu.sync_copy(parts[0].at[sl], seg_v)
    for p in range(1, n):
        pltpu.sync_copy(parts[p].at[sl], tmp_v)
        for k in range(SEG // 16):
            s = pl.ds(k * 16, 16)
            seg_v[s] = seg_v[s] + tmp_v[s]


@functools.partial(
    pl.kernel,
    out_type=[jax.ShapeDtypeStruct((NP,), jnp.float32) for _ in range(4)],
    mesh=_mesh,
    compiler_params=_params,
    scratch_types=[
        pltpu.VMEM((R, C), jnp.int32),
        pltpu.VMEM((R, C), jnp.int32),
        pltpu.VMEM((C,), jnp.float32),
        pltpu.VMEM((SEG,), jnp.float32),
        pltpu.VMEM((SEG,), jnp.float32),
    ]
    + [pltpu.VMEM_SHARED((NP,), jnp.float32) for _ in range(2 * P)]
    + [pltpu.SemaphoreType.DMA],
)
def _k1(es_hbm, do0_hbm, di0_hbm, do1_hbm, di1_hbm,
        src_v, dst_v, ones_v, seg_v, tmp_v, *rest):
    do_sp = rest[:P]
    di_sp = rest[P:2 * P]
    sem = rest[2 * P]
    cid = lax.axis_index("c")
    sid = lax.axis_index("s")
    wid = sid * NC + cid

    _zero_fill(seg_v, SEG)
    for k in range(C // 16):
        ones_v[pl.ds(k * 16, 16)] = jnp.ones((16,), jnp.float32)

    pltpu.sync_copy(es_hbm.at[0, wid], src_v)
    pltpu.sync_copy(es_hbm.at[1, wid], dst_v)
    for p in range(P):
        pltpu.sync_copy(seg_v, do_sp[p].at[pl.ds(sid * SEG, SEG)])
        pltpu.sync_copy(seg_v, di_sp[p].at[pl.ds(sid * SEG, SEG)])
    plsc.subcore_barrier()

    def quad(t, _):
        @pl.when(t > 0)
        def _():
            pltpu.make_async_copy(do0_hbm.at[pl.ds(0, SEG)], seg_v,
                                  sem).wait()

        j0 = t * P
        for p in range(P):
            pltpu.async_copy(ones_v, do_sp[p].at[src_v.at[j0 + p]], sem,
                             add=True)
            pltpu.async_copy(ones_v, di_sp[p].at[dst_v.at[j0 + p]], sem,
                             add=True)
        return 0

    NQ = R // P
    lax.fori_loop(0, NQ, quad, 0)
    pltpu.make_async_copy(do0_hbm.at[pl.ds(0, SEG)], seg_v, sem).wait()
    for j in range(NQ * P, R):
        pltpu.async_copy(ones_v, do_sp[0].at[src_v.at[j]], sem, add=True)
        pltpu.async_copy(ones_v, di_sp[0].at[dst_v.at[j]], sem, add=True)
        pltpu.make_async_copy(do0_hbm.at[pl.ds(0, 2 * C)],
                              seg_v.at[pl.ds(0, 2 * C)], sem).wait()
    plsc.subcore_barrier()

    sl = pl.ds(sid * SEG, SEG)

    @pl.when(cid == 0)
    def _():
        _acc_seg(seg_v, tmp_v, do_sp, sl)
        pltpu.sync_copy(seg_v, do0_hbm.at[sl])
        _acc_seg(seg_v, tmp_v, di_sp, sl)
        pltpu.sync_copy(seg_v, di0_hbm.at[sl])

    @pl.when(cid == 1)
    def _():
        _acc_seg(seg_v, tmp_v, do_sp, sl)
        pltpu.sync_copy(seg_v, do1_hbm.at[sl])
        _acc_seg(seg_v, tmp_v, di_sp, sl)
        pltpu.sync_copy(seg_v, di1_hbm.at[sl])


def _k2a_body(x_ref, w_ref, xw_ref):
    xw_ref[...] = jnp.sum(x_ref[...] * w_ref[...], axis=-1)


_k2a = pl.pallas_call(
    _k2a_body,
    out_shape=jax.ShapeDtypeStruct((N,), jnp.float32),
)


def _k2b_body(xw_ref, do0_ref, di0_ref, do1_ref, di1_ref, h_ref, nd_ref):
    deg_out = do0_ref[pl.ds(0, N)] + do1_ref[pl.ds(0, N)]
    deg_in = di0_ref[pl.ds(0, N)] + di1_ref[pl.ds(0, N)]
    h_ref[...] = xw_ref[...] * lax.rsqrt(jnp.maximum(deg_out, 1.0))
    nd_ref[...] = lax.rsqrt(jnp.maximum(deg_in, 1.0))


_k2b = pl.pallas_call(
    _k2b_body,
    out_shape=(
        jax.ShapeDtypeStruct((N,), jnp.float32),
        jax.ShapeDtypeStruct((N,), jnp.float32),
    ),
)


@functools.partial(
    pl.kernel,
    out_type=[jax.ShapeDtypeStruct((NP,), jnp.float32) for _ in range(2)],
    mesh=_mesh,
    compiler_params=_params,
    scratch_types=[
        pltpu.VMEM((R, C), jnp.int32),
        pltpu.VMEM((R, C), jnp.int32),
        pltpu.VMEM((R, C), jnp.float32),
        pltpu.VMEM((N,), jnp.float32),
        pltpu.VMEM((SEG,), jnp.float32),
        pltpu.VMEM((SEG,), jnp.float32),
    ]
    + [pltpu.VMEM_SHARED((NP,), jnp.float32) for _ in range(2 * P)]
    + [pltpu.SemaphoreType.DMA],
)
def _k3(es_hbm, h_hbm, a0_hbm, a1_hbm,
        src_v, dst_v, vals_v, h_v, seg_v, tmp_v, *rest):
    agg_sp = rest[:2 * P]
    sem = rest[2 * P]
    cid = lax.axis_index("c")
    sid = lax.axis_index("s")
    wid = sid * NC + cid

    _zero_fill(seg_v, SEG)
    pltpu.sync_copy(es_hbm.at[0, wid], src_v)
    pltpu.sync_copy(es_hbm.at[1, wid], dst_v)
    pltpu.sync_copy(h_hbm, h_v)
    for p in range(2 * P):
        pltpu.sync_copy(seg_v, agg_sp[p].at[pl.ds(sid * SEG, SEG)])
    plsc.subcore_barrier()

    def quad(t, _):
        @pl.when(t > 1)
        def _():
            pltpu.make_async_copy(h_hbm.at[pl.ds(0, P * C)],
                                  seg_v.at[pl.ds(0, P * C)], sem).wait()

        j0 = t * P
        grp = lax.rem(t, 2) * P
        for p in range(P):
            j = j0 + p
            for k in range(C // 16):
                idx16 = src_v[j, pl.ds(k * 16, 16)]
                vals_v[j, pl.ds(k * 16, 16)] = plsc.load_gather(h_v, [idx16])

            @pl.when(grp == 0)
            def _(p=p, j=j):
                pltpu.async_copy(vals_v.at[j], agg_sp[p].at[dst_v.at[j]],
                                 sem, add=True)

            @pl.when(grp == P)
            def _(p=p, j=j):
                pltpu.async_copy(vals_v.at[j], agg_sp[P + p].at[dst_v.at[j]],
                                 sem, add=True)
        return 0

    NQ = R // P
    lax.fori_loop(0, NQ, quad, 0)
    pltpu.make_async_copy(h_hbm.at[pl.ds(0, 2 * P * C)],
                          seg_v.at[pl.ds(0, 2 * P * C)], sem).wait()
    for j in range(NQ * P, R):
        for k in range(C // 16):
            idx16 = src_v[j, pl.ds(k * 16, 16)]
            vals_v[j, pl.ds(k * 16, 16)] = plsc.load_gather(h_v, [idx16])
        pltpu.async_copy(vals_v.at[j], agg_sp[0].at[dst_v.at[j]], sem,
                         add=True)
        pltpu.make_async_copy(h_hbm.at[pl.ds(0, C)],
                              seg_v.at[pl.ds(0, C)], sem).wait()
    plsc.subcore_barrier()

    sl = pl.ds(sid * SEG, SEG)
    _acc_seg(seg_v, tmp_v, agg_sp, sl, 2 * P)

    @pl.when(cid == 0)
    def _():
        pltpu.sync_copy(seg_v, a0_hbm.at[sl])

    @pl.when(cid == 1)
    def _():
        pltpu.sync_copy(seg_v, a1_hbm.at[sl])


def _k4_body(a0_ref, a1_ref, nd_ref, b_ref, out_ref):
    agg = a0_ref[pl.ds(0, N)] + a1_ref[pl.ds(0, N)]
    out_ref[...] = jnp.maximum(agg * nd_ref[...] + b_ref[0, 0], 0.0)


_k4 = pl.pallas_call(
    _k4_body,
    out_shape=jax.ShapeDtypeStruct((N,), jnp.float32),
)


def kernel(x, edge_index, W_mat, b):
    es = edge_index.reshape(2, NW, R, C)
    wr = W_mat.reshape(1, D)

    xw = _k2a(x, wr)
    do0, di0, do1, di1 = _k1(es)
    h, nd = _k2b(xw, do0, di0, do1, di1)

    a0, a1 = _k3(es, h)
    o = _k4(a0, a1, nd, b.reshape(1, 1))
    return o.reshape(N, 1)
